# baseline (speedup 1.0000x reference)
import functools

import jax
import jax.numpy as jnp
from jax.experimental import pallas as pl
from jax.experimental.pallas import tpu as pltpu

_E = 8
_LANES = 128
_NEG_INF = float("-inf")


def _moe_body(xb_ref, gwt_ref, a_ref, b_ref, out_ref, sel0_ref,
              sel1_ref, w0_ref, w1_ref):
    e = pl.program_id(0)

    @pl.when(e == 0)
    def _routing():
        logits = jnp.dot(xb_ref[...], gwt_ref[...],
                         preferred_element_type=jnp.float32)
        lane = jax.lax.broadcasted_iota(jnp.int32, logits.shape, 1)
        valid = lane < _E
        logits = jnp.where(valid, logits, _NEG_INF)
        m1 = jnp.max(logits, axis=1, keepdims=True)
        idx0 = jnp.min(jnp.where(logits == m1, lane, _LANES), axis=1,
                       keepdims=True)
        logits2 = jnp.where(lane == idx0, _NEG_INF, logits)
        m2 = jnp.max(logits2, axis=1, keepdims=True)
        idx1 = jnp.min(jnp.where(logits2 == m2, lane, _LANES), axis=1,
                       keepdims=True)
        s = jnp.exp(m2 - m1)
        w0 = 1.0 / (1.0 + s)
        sel0_ref[...] = idx0.astype(jnp.float32)
        sel1_ref[...] = idx1.astype(jnp.float32)
        w0_ref[...] = w0
        w1_ref[...] = 1.0 - w0

    ef = e.astype(jnp.float32)
    w_col = (jnp.where(sel0_ref[...] == ef, w0_ref[...], 0.0)
             + jnp.where(sel1_ref[...] == ef, w1_ref[...], 0.0))
    y = jnp.dot(xb_ref[...], a_ref[0],
                preferred_element_type=jnp.float32)
    contrib = w_col * y + w_col * b_ref[0]

    @pl.when(e == 0)
    def _init():
        out_ref[...] = contrib

    @pl.when(e != 0)
    def _acc():
        out_ref[...] += contrib


@functools.partial(jax.jit, static_argnames=())
def kernel(inputs, gate_w, expert_A, expert_b):
    batch_shape = inputs.shape[:-1]
    d = inputs.shape[-1]
    x = inputs.reshape(-1, d)
    t = x.shape[0]

    gwt = (jnp.zeros((d, _LANES), dtype=jnp.bfloat16)
           .at[:, :_E].set(gate_w.T.astype(jnp.bfloat16)))
    xb = x.astype(jnp.bfloat16)
    ab = expert_A.astype(jnp.bfloat16)

    out = pl.pallas_call(
        _moe_body,
        grid=(_E,),
        in_specs=[
            pl.BlockSpec((t, d), lambda e: (0, 0)),
            pl.BlockSpec((d, _LANES), lambda e: (0, 0)),
            pl.BlockSpec((1, d, d), lambda e: (e, 0, 0)),
            pl.BlockSpec((1, 1, d), lambda e: (e, 0, 0)),
        ],
        out_specs=pl.BlockSpec((t, d), lambda e: (0, 0)),
        out_shape=jax.ShapeDtypeStruct((t, d), jnp.float32),
        scratch_shapes=[
            pltpu.VMEM((t, 1), jnp.float32),
            pltpu.VMEM((t, 1), jnp.float32),
            pltpu.VMEM((t, 1), jnp.float32),
            pltpu.VMEM((t, 1), jnp.float32),
        ],
    )(xb, gwt, ab, expert_b.reshape(_E, 1, d))
    return out.reshape(*batch_shape, d)


# speedup vs baseline: 1.2948x; 1.2948x over previous
import functools

import jax
import jax.numpy as jnp
from jax.experimental import pallas as pl
from jax.experimental.pallas import tpu as pltpu

_E = 8
_LANES = 128
_NEG_INF = float("-inf")
_TILE = 256


def _moe_body(x_ref, gwt_ref, a_ref, bpad_ref, out_ref, xw_ref):
    xt = x_ref[...]
    logits = jnp.dot(xt, gwt_ref[...],
                     preferred_element_type=jnp.float32)
    lane = jax.lax.broadcasted_iota(jnp.int32, logits.shape, 1)
    logits = jnp.where(lane < _E, logits, _NEG_INF)
    m1 = jnp.max(logits, axis=1, keepdims=True)
    idx0 = jnp.min(jnp.where(logits == m1, lane, _LANES), axis=1,
                   keepdims=True)
    logits2 = jnp.where(lane == idx0, _NEG_INF, logits)
    m2 = jnp.max(logits2, axis=1, keepdims=True)
    idx1 = jnp.min(jnp.where(logits2 == m2, lane, _LANES), axis=1,
                   keepdims=True)
    s = jnp.exp(m2 - m1)
    w0 = 1.0 / (1.0 + s)
    w1 = 1.0 - w0
    dense_w = (jnp.where(lane == idx0, w0, 0.0)
               + jnp.where(lane == idx1, w1, 0.0))
    d = xt.shape[1]
    for e in range(_E):
        w_col = (jnp.where(idx0 == e, w0, 0.0)
                 + jnp.where(idx1 == e, w1, 0.0))
        xw_ref[:, e * d:(e + 1) * d] = w_col * xt
    out_ref[...] = (
        jnp.dot(xw_ref[...], a_ref[...], preferred_element_type=jnp.float32)
        + jnp.dot(dense_w, bpad_ref[...], preferred_element_type=jnp.float32))


@functools.partial(jax.jit, static_argnames=())
def kernel(inputs, gate_w, expert_A, expert_b):
    batch_shape = inputs.shape[:-1]
    d = inputs.shape[-1]
    x = inputs.reshape(-1, d)
    t = x.shape[0]

    gwt = jnp.zeros((d, _LANES), dtype=gate_w.dtype).at[:, :_E].set(gate_w.T)
    a_stack = expert_A.reshape(_E * d, d)
    bpad = jnp.zeros((_LANES, d), dtype=expert_b.dtype).at[:_E].set(expert_b)

    out = pl.pallas_call(
        _moe_body,
        grid=(t // _TILE,),
        in_specs=[
            pl.BlockSpec((_TILE, d), lambda i: (i, 0)),
            pl.BlockSpec((d, _LANES), lambda i: (0, 0)),
            pl.BlockSpec((_E * d, d), lambda i: (0, 0)),
            pl.BlockSpec((_LANES, d), lambda i: (0, 0)),
        ],
        out_specs=pl.BlockSpec((_TILE, d), lambda i: (i, 0)),
        out_shape=jax.ShapeDtypeStruct((t, d), jnp.float32),
        scratch_shapes=[
            pltpu.VMEM((_TILE, _E * d), jnp.float32),
        ],
    )(x, gwt, a_stack, bpad)
    return out.reshape(*batch_shape, d)


# speedup vs baseline: 1.3940x; 1.0766x over previous
import functools

import jax
import jax.numpy as jnp
from jax.experimental import pallas as pl
from jax.experimental.pallas import tpu as pltpu

_E = 8
_LANES = 128
_NEG_INF = float("-inf")
_TILE = 1024


def _moe_body(x_ref, gwt_ref, a_ref, bpad_ref, out_ref, xw_ref):
    xt = x_ref[...]
    logits = jnp.dot(xt, gwt_ref[...],
                     preferred_element_type=jnp.float32)
    lane = jax.lax.broadcasted_iota(jnp.int32, logits.shape, 1)
    logits = jnp.where(lane < _E, logits, _NEG_INF)
    m1 = jnp.max(logits, axis=1, keepdims=True)
    idx0 = jnp.min(jnp.where(logits == m1, lane, _LANES), axis=1,
                   keepdims=True)
    logits2 = jnp.where(lane == idx0, _NEG_INF, logits)
    m2 = jnp.max(logits2, axis=1, keepdims=True)
    idx1 = jnp.min(jnp.where(logits2 == m2, lane, _LANES), axis=1,
                   keepdims=True)
    s = jnp.exp(m2 - m1)
    w0 = 1.0 / (1.0 + s)
    w1 = 1.0 - w0
    dense_w = (jnp.where(lane == idx0, w0, 0.0)
               + jnp.where(lane == idx1, w1, 0.0))
    d = xt.shape[1]
    for e in range(_E):
        w_col = (jnp.where(idx0 == e, w0, 0.0)
                 + jnp.where(idx1 == e, w1, 0.0))
        xw_ref[:, e * d:(e + 1) * d] = (w_col * xt).astype(jnp.bfloat16)
    out_ref[...] = (
        jnp.dot(xw_ref[...], a_ref[...], preferred_element_type=jnp.float32)
        + jnp.dot(dense_w, bpad_ref[...], preferred_element_type=jnp.float32))


@functools.partial(jax.jit, static_argnames=())
def kernel(inputs, gate_w, expert_A, expert_b):
    batch_shape = inputs.shape[:-1]
    d = inputs.shape[-1]
    x = inputs.reshape(-1, d)
    t = x.shape[0]

    gwt = jnp.zeros((d, _LANES), dtype=gate_w.dtype).at[:, :_E].set(gate_w.T)
    a_stack = expert_A.reshape(_E * d, d)
    bpad = jnp.zeros((_LANES, d), dtype=expert_b.dtype).at[:_E].set(expert_b)

    out = pl.pallas_call(
        _moe_body,
        grid=(t // _TILE,),
        in_specs=[
            pl.BlockSpec((_TILE, d), lambda i: (i, 0)),
            pl.BlockSpec((d, _LANES), lambda i: (0, 0)),
            pl.BlockSpec((_E * d, d), lambda i: (0, 0)),
            pl.BlockSpec((_LANES, d), lambda i: (0, 0)),
        ],
        out_specs=pl.BlockSpec((_TILE, d), lambda i: (i, 0)),
        out_shape=jax.ShapeDtypeStruct((t, d), jnp.float32),
        scratch_shapes=[
            pltpu.VMEM((_TILE, _E * d), jnp.bfloat16),
        ],
    )(x, gwt, a_stack, bpad)
    return out.reshape(*batch_shape, d)
